# Initial kernel scaffold; baseline (speedup 1.0000x reference)
import jax
import jax.numpy as jnp
from jax.experimental import pallas as pl


def kernel(x, Wq, Wk, Wres, bres):
    raise NotImplementedError("write your pallas kernel here")


# speedup vs baseline: 8.2791x; 8.2791x over previous
import jax
import jax.numpy as jnp
from jax.experimental import pallas as pl
from jax.experimental.pallas import tpu as pltpu

_T = 512
_EPS = 1e-4


def _hnet_body(x_ref, wq_ref, wk_ref, wres_ref, bres_ref, o_ref, carry_ref):
    i = pl.program_id(1)
    T = x_ref.shape[1]
    D = x_ref.shape[2]

    xb = x_ref[0]
    qq = jnp.dot(xb, wq_ref[...], preferred_element_type=jnp.float32)
    kk = jnp.dot(xb, wk_ref[...], preferred_element_type=jnp.float32)
    rr = jnp.dot(xb, wres_ref[...], preferred_element_type=jnp.float32)
    rr = rr + bres_ref[...]

    @pl.when(i == 0)
    def _():
        carry_ref[...] = jnp.zeros_like(carry_ref)

    z_carry = carry_ref[0:1, :]
    q_carry = carry_ref[1:2, :]

    row = jax.lax.broadcasted_iota(jnp.int32, (T, 1), 0)

    q_shift = jnp.where(row == 0, q_carry, pltpu.roll(qq, 1, axis=0))

    qn2 = jnp.sum(q_shift * q_shift, axis=1, keepdims=True)
    kn2 = jnp.sum(kk * kk, axis=1, keepdims=True)
    qk = jnp.sum(q_shift * kk, axis=1, keepdims=True)
    denom = jnp.maximum(jnp.sqrt(qn2), 1e-8) * jnp.maximum(jnp.sqrt(kn2), 1e-8)
    cos = qk / denom
    p = jnp.clip(0.5 - 0.5 * cos, 0.0, 1.0)

    p = jnp.where((i == 0) & (row == 0), 1.0, p)

    sel = p >= 0.5
    p_eff = jnp.where(sel, jnp.clip(p, _EPS, 1.0 - _EPS), 0.0)

    a = 1.0 - p_eff
    bv = p_eff * xb

    d = 1
    while d < T:
        pred = row >= d
        a_sh = jnp.where(pred, pltpu.roll(a, d, axis=0), 1.0)
        bv_sh = jnp.where(pred, pltpu.roll(bv, d, axis=0), 0.0)
        bv = a * bv_sh + bv
        a = a * a_sh
        d *= 2

    z = bv + a * z_carry

    o_ref[0] = rr + z

    carry_ref[0:1, :] = z[T - 1:T, :]
    carry_ref[1:2, :] = qq[T - 1:T, :]


def kernel(x, Wq, Wk, Wres, bres):
    B, L, D = x.shape
    T = _T
    grid = (B, L // T)
    out = pl.pallas_call(
        _hnet_body,
        grid=grid,
        in_specs=[
            pl.BlockSpec((1, T, D), lambda b, i: (b, i, 0)),
            pl.BlockSpec((D, D), lambda b, i: (0, 0)),
            pl.BlockSpec((D, D), lambda b, i: (0, 0)),
            pl.BlockSpec((D, D), lambda b, i: (0, 0)),
            pl.BlockSpec((1, D), lambda b, i: (0, 0)),
        ],
        out_specs=pl.BlockSpec((1, T, D), lambda b, i: (b, i, 0)),
        out_shape=jax.ShapeDtypeStruct((B, L, D), jnp.float32),
        scratch_shapes=[pltpu.VMEM((2, D), jnp.float32)],
        compiler_params=pltpu.CompilerParams(
            dimension_semantics=("parallel", "arbitrary"),
        ),
    )(x, Wq, Wk, Wres, bres.reshape(1, D))
    return out
